# baseline (speedup 1.0000x reference)
import jax
import jax.numpy as jnp
from jax import lax
from jax.experimental import pallas as pl
from jax.experimental.pallas import tpu as pltpu
from jax.experimental.pallas import tpu_sc as plsc

INSN = 12
NINSN = 10
SEQ = 120
D = 64
NGRP = 3
OUT_PER_ROW = NGRP * NINSN

NC, NS = 2, 16
NW = NC * NS

B = 4096
ROWS_PER_W = B // NW
BBLK = 32
NBLK = ROWS_PER_W // BBLK
GROWS = BBLK * INSN
WIDX = ROWS_PER_W * SEQ


def _body(idx_hbm, table_hbm, pos_hbm, out_hbm,
          idx_v, rows_v0, rows_v1, out_v0, out_v1, pos_v,
          gsem0, gsem1, osem0, osem1):
    wid = lax.axis_index("s") * NC + lax.axis_index("c")
    rows_v = (rows_v0, rows_v1)
    out_v = (out_v0, out_v1)
    gsem = (gsem0, gsem1)
    osem = (osem0, osem1)

    pltpu.sync_copy(pos_hbm, pos_v)
    pltpu.sync_copy(idx_hbm.at[pl.ds(wid * WIDX, WIDX)], idx_v)

    i16 = lax.iota(jnp.int32, 16)
    dh_vec = [(q * 16 + i16) >> 3 for q in range(4)]
    dl_vec = [(q * 16 + i16) & 7 for q in range(4)]
    g_vec = [jnp.full((16,), g, jnp.int32) for g in range(NGRP)]

    def gather_start(p, buf):
        pltpu.async_copy(
            table_hbm.at[idx_v.at[pl.ds(p * GROWS, GROWS)]],
            rows_v[buf], gsem[buf])

    def gather_wait(buf):
        pltpu.make_async_copy(
            table_hbm.at[idx_v.at[pl.ds(0, GROWS)]],
            rows_v[buf], gsem[buf]).wait()

    def out_dst(j):
        return out_hbm.at[pl.ds(NGRP * j, NGRP), :, wid, :, :]

    gather_start(0, 0)

    for j in range(NINSN):
        oj = j % 2
        ob = out_v[oj]
        if j >= 2:
            pltpu.make_async_copy(out_v[oj], out_dst(j), osem[oj]).wait()
        pos_q = [pos_v[j, pl.ds(q * 16, 16)] for q in range(4)]

        for blk in range(NBLK):
            p = j * NBLK + blk
            if p + 1 < NINSN * NBLK:
                gather_start(p + 1, (blk + 1) % 2)
            gather_wait(blk % 2)
            rows = rows_v[blk % 2]

            @pl.loop(0, BBLK)
            def brow(b32):
                ro = b32 * INSN
                b_bc = jnp.full((16,), blk * BBLK, jnp.int32) + b32
                for q in range(4):
                    sl = pl.ds(q * 16, 16)
                    a1 = rows[ro + 0, sl] + rows[ro + 1, sl]
                    a2 = ((rows[ro + 2, sl] + rows[ro + 3, sl])
                          + (rows[ro + 4, sl] + rows[ro + 5, sl])
                          + rows[ro + 6, sl])
                    a3 = ((rows[ro + 7, sl] + rows[ro + 8, sl])
                          + (rows[ro + 9, sl] + rows[ro + 10, sl])
                          + rows[ro + 11, sl])
                    for g, a in ((0, a1), (1, a2), (2, a3)):
                        plsc.store_scatter(
                            ob, [g_vec[g], dh_vec[q], dl_vec[q], b_bc],
                            a + pos_q[q])

        pltpu.async_copy(out_v[oj], out_dst(j), osem[oj])

    for oj in range(2):
        pltpu.make_async_copy(out_v[oj], out_dst(NINSN - 2 + oj),
                              osem[oj]).wait()


@jax.jit
def _run(idx_perm, word_table, pos10):
    mesh = plsc.VectorSubcoreMesh(
        core_axis_name="c", subcore_axis_name="s", num_cores=NC, num_subcores=NS)
    k = pl.kernel(
        _body,
        out_type=jax.ShapeDtypeStruct((OUT_PER_ROW, 8, NW, 8, 128),
                                      jnp.float32),
        mesh=mesh,
        scratch_types=[
            pltpu.VMEM((WIDX,), jnp.int32),
            pltpu.VMEM((GROWS, D), jnp.float32),
            pltpu.VMEM((GROWS, D), jnp.float32),
            pltpu.VMEM((NGRP, 8, 8, 128), jnp.float32),
            pltpu.VMEM((NGRP, 8, 8, 128), jnp.float32),
            pltpu.VMEM((NINSN, D), jnp.float32),
            pltpu.SemaphoreType.DMA,
            pltpu.SemaphoreType.DMA,
            pltpu.SemaphoreType.DMA,
            pltpu.SemaphoreType.DMA,
        ],
        compiler_params=pltpu.CompilerParams(
            use_tc_tiling_on_sc=False, needs_layout_passes=False),
    )
    return k(idx_perm, word_table, pos10)


def kernel(word, word_table, pos_table):
    idx_perm = (word.astype(jnp.int32)
                .reshape(NW, NBLK, BBLK, NINSN, INSN)
                .transpose(0, 3, 1, 2, 4)
                .reshape(-1))
    pos10 = lax.slice_in_dim(pos_table, 1, 1 + NINSN, axis=0)
    out5 = _run(idx_perm, word_table, pos10)
    return out5.transpose(2, 4, 0, 1, 3).reshape(B, OUT_PER_ROW, D)


# speedup vs baseline: 1.0024x; 1.0024x over previous
import jax
import jax.numpy as jnp
from jax import lax
from jax.experimental import pallas as pl
from jax.experimental.pallas import tpu as pltpu
from jax.experimental.pallas import tpu_sc as plsc

INSN = 12
NINSN = 10
SEQ = 120
D = 64
NGRP = 3
OUT_PER_ROW = NGRP * NINSN

NC, NS = 2, 16
NW = NC * NS

B = 4096
ROWS_PER_W = B // NW
BBLK = 32
NBLK = ROWS_PER_W // BBLK
GROWS = BBLK * INSN
WIDX = ROWS_PER_W * SEQ


def _body(idx_hbm, table_hbm, pos_hbm, out_hbm,
          idx_v0, idx_v1, rows_v0, rows_v1, out_v0, out_v1, pos_v,
          gsem0, gsem1, osem0, osem1, isem0, isem1):
    wid = lax.axis_index("s") * NC + lax.axis_index("c")
    idx_v = (idx_v0, idx_v1)
    rows_v = (rows_v0, rows_v1)
    out_v = (out_v0, out_v1)
    gsem = (gsem0, gsem1)
    osem = (osem0, osem1)
    isem = (isem0, isem1)

    pltpu.sync_copy(pos_hbm, pos_v)
    ibase = wid * WIDX

    def idx_start(p):
        pltpu.async_copy(idx_hbm.at[pl.ds(ibase + p * GROWS, GROWS)],
                         idx_v[p % 2], isem[p % 2])

    def idx_wait(p):
        pltpu.make_async_copy(idx_hbm.at[pl.ds(ibase, GROWS)],
                              idx_v[p % 2], isem[p % 2]).wait()

    i16 = lax.iota(jnp.int32, 16)
    dh_vec = [(q * 16 + i16) >> 3 for q in range(4)]
    dl_vec = [(q * 16 + i16) & 7 for q in range(4)]
    g_vec = [jnp.full((16,), g, jnp.int32) for g in range(NGRP)]

    def gather_start(p):
        pltpu.async_copy(table_hbm.at[idx_v[p % 2]],
                         rows_v[p % 2], gsem[p % 2])

    def gather_wait(p):
        pltpu.make_async_copy(table_hbm.at[idx_v[p % 2]],
                              rows_v[p % 2], gsem[p % 2]).wait()

    def out_dst(j):
        return out_hbm.at[pl.ds(NGRP * j, NGRP), :, wid, :, :]

    pltpu.sync_copy(idx_hbm.at[pl.ds(ibase, GROWS)], idx_v[0])
    idx_start(1)
    gather_start(0)

    for j in range(NINSN):
        oj = j % 2
        ob = out_v[oj]
        if j >= 2:
            pltpu.make_async_copy(out_v[oj], out_dst(j), osem[oj]).wait()
        pos_q = [pos_v[j, pl.ds(q * 16, 16)] for q in range(4)]

        for blk in range(NBLK):
            p = j * NBLK + blk
            if p + 1 < NINSN * NBLK:
                idx_wait(p + 1)
                gather_start(p + 1)
            gather_wait(p)
            if p + 2 < NINSN * NBLK:
                idx_start(p + 2)
            rows = rows_v[p % 2]

            @pl.loop(0, BBLK)
            def brow(b32):
                ro = b32 * INSN
                b_bc = jnp.full((16,), blk * BBLK, jnp.int32) + b32
                for q in range(4):
                    sl = pl.ds(q * 16, 16)
                    a1 = rows[ro + 0, sl] + rows[ro + 1, sl]
                    a2 = ((rows[ro + 2, sl] + rows[ro + 3, sl])
                          + (rows[ro + 4, sl] + rows[ro + 5, sl])
                          + rows[ro + 6, sl])
                    a3 = ((rows[ro + 7, sl] + rows[ro + 8, sl])
                          + (rows[ro + 9, sl] + rows[ro + 10, sl])
                          + rows[ro + 11, sl])
                    for g, a in ((0, a1), (1, a2), (2, a3)):
                        plsc.store_scatter(
                            ob, [g_vec[g], dh_vec[q], dl_vec[q], b_bc],
                            a + pos_q[q])

        pltpu.async_copy(out_v[oj], out_dst(j), osem[oj])

    for oj in range(2):
        pltpu.make_async_copy(out_v[oj], out_dst(NINSN - 2 + oj),
                              osem[oj]).wait()


@jax.jit
def _run(idx_perm, word_table, pos10):
    mesh = plsc.VectorSubcoreMesh(
        core_axis_name="c", subcore_axis_name="s", num_cores=NC, num_subcores=NS)
    k = pl.kernel(
        _body,
        out_type=jax.ShapeDtypeStruct((OUT_PER_ROW, 8, NW, 8, 128),
                                      jnp.float32),
        mesh=mesh,
        scratch_types=[
            pltpu.VMEM((GROWS,), jnp.int32),
            pltpu.VMEM((GROWS,), jnp.int32),
            pltpu.VMEM((GROWS, D), jnp.float32),
            pltpu.VMEM((GROWS, D), jnp.float32),
            pltpu.VMEM((NGRP, 8, 8, 128), jnp.float32),
            pltpu.VMEM((NGRP, 8, 8, 128), jnp.float32),
            pltpu.VMEM((NINSN, D), jnp.float32),
            pltpu.SemaphoreType.DMA,
            pltpu.SemaphoreType.DMA,
            pltpu.SemaphoreType.DMA,
            pltpu.SemaphoreType.DMA,
            pltpu.SemaphoreType.DMA,
            pltpu.SemaphoreType.DMA,
        ],
        compiler_params=pltpu.CompilerParams(
            use_tc_tiling_on_sc=False, needs_layout_passes=False),
    )
    return k(idx_perm, word_table, pos10)


def kernel(word, word_table, pos_table):
    idx_perm = (word.astype(jnp.int32)
                .reshape(NW, NBLK, BBLK, NINSN, INSN)
                .transpose(0, 3, 1, 2, 4)
                .reshape(-1))
    pos10 = lax.slice_in_dim(pos_table, 1, 1 + NINSN, axis=0)
    out5 = _run(idx_perm, word_table, pos10)
    return out5.transpose(2, 4, 0, 1, 3).reshape(B, OUT_PER_ROW, D)


# speedup vs baseline: 1.4826x; 1.4790x over previous
import jax
import jax.numpy as jnp
from jax import lax
from jax.experimental import pallas as pl
from jax.experimental.pallas import tpu as pltpu
from jax.experimental.pallas import tpu_sc as plsc

INSN = 12
NINSN = 10
SEQ = 120
D = 64
NGRP = 3
OUT_PER_ROW = NGRP * NINSN

NC, NS = 2, 16
NW = NC * NS

B = 4096
ROWS_PER_W = B // NW
BBLK = 32
NBLK = ROWS_PER_W // BBLK
GROWS = BBLK * INSN
WIDX = ROWS_PER_W * SEQ
BPAD = 129


def _body(idx_hbm, table_hbm, pos_hbm, out_hbm,
          idx_v0, idx_v1, rows_v0, rows_v1, out_v0, out_v1, pos_v,
          gsem0, gsem1, osem0, osem1, isem0, isem1):
    wid = lax.axis_index("s") * NC + lax.axis_index("c")
    idx_v = (idx_v0, idx_v1)
    rows_v = (rows_v0, rows_v1)
    out_v = (out_v0, out_v1)
    gsem = (gsem0, gsem1)
    osem = (osem0, osem1)
    isem = (isem0, isem1)

    pltpu.sync_copy(pos_hbm, pos_v)
    ibase = wid * WIDX

    def idx_start(p):
        pltpu.async_copy(idx_hbm.at[pl.ds(ibase + p * GROWS, GROWS)],
                         idx_v[p % 2], isem[p % 2])

    def idx_wait(p):
        pltpu.make_async_copy(idx_hbm.at[pl.ds(ibase, GROWS)],
                              idx_v[p % 2], isem[p % 2]).wait()

    i16 = lax.iota(jnp.int32, 16)
    dh_vec = [(q * 16 + i16) >> 3 for q in range(4)]
    dl_vec = [(q * 16 + i16) & 7 for q in range(4)]
    g_vec = [jnp.full((16,), g, jnp.int32) for g in range(NGRP)]

    def gather_start(p):
        pltpu.async_copy(table_hbm.at[idx_v[p % 2]],
                         rows_v[p % 2], gsem[p % 2])

    def gather_wait(p):
        pltpu.make_async_copy(table_hbm.at[idx_v[p % 2]],
                              rows_v[p % 2], gsem[p % 2]).wait()

    def out_dst(j):
        return out_hbm.at[pl.ds(NGRP * j, NGRP), :, wid, :, :]

    pltpu.sync_copy(idx_hbm.at[pl.ds(ibase, GROWS)], idx_v[0])
    idx_start(1)
    gather_start(0)

    for j in range(NINSN):
        oj = j % 2
        ob = out_v[oj]
        if j >= 2:
            pltpu.make_async_copy(ob.at[:, :, :, pl.ds(0, 128)],
                                  out_dst(j), osem[oj]).wait()
        pos_q = [pos_v[j, pl.ds(q * 16, 16)] for q in range(4)]

        for blk in range(NBLK):
            p = j * NBLK + blk
            if p + 1 < NINSN * NBLK:
                idx_wait(p + 1)
                gather_start(p + 1)
            gather_wait(p)
            if p + 2 < NINSN * NBLK:
                idx_start(p + 2)
            rows = rows_v[p % 2]

            @pl.loop(0, BBLK)
            def brow(b32):
                ro = b32 * INSN
                b_bc = jnp.full((16,), blk * BBLK, jnp.int32) + b32
                for q in range(4):
                    sl = pl.ds(q * 16, 16)
                    a1 = rows[ro + 0, sl] + rows[ro + 1, sl]
                    a2 = ((rows[ro + 2, sl] + rows[ro + 3, sl])
                          + (rows[ro + 4, sl] + rows[ro + 5, sl])
                          + rows[ro + 6, sl])
                    a3 = ((rows[ro + 7, sl] + rows[ro + 8, sl])
                          + (rows[ro + 9, sl] + rows[ro + 10, sl])
                          + rows[ro + 11, sl])
                    for g, a in ((0, a1), (1, a2), (2, a3)):
                        plsc.store_scatter(
                            ob, [g_vec[g], dh_vec[q], dl_vec[q], b_bc],
                            a + pos_q[q])

        pltpu.async_copy(ob.at[:, :, :, pl.ds(0, 128)], out_dst(j), osem[oj])

    for oj in range(2):
        pltpu.make_async_copy(out_v[oj].at[:, :, :, pl.ds(0, 128)],
                              out_dst(NINSN - 2 + oj), osem[oj]).wait()


@jax.jit
def _run(idx_perm, word_table, pos10):
    mesh = plsc.VectorSubcoreMesh(
        core_axis_name="c", subcore_axis_name="s", num_cores=NC, num_subcores=NS)
    k = pl.kernel(
        _body,
        out_type=jax.ShapeDtypeStruct((OUT_PER_ROW, 8, NW, 8, 128),
                                      jnp.float32),
        mesh=mesh,
        scratch_types=[
            pltpu.VMEM((GROWS,), jnp.int32),
            pltpu.VMEM((GROWS,), jnp.int32),
            pltpu.VMEM((GROWS, D), jnp.float32),
            pltpu.VMEM((GROWS, D), jnp.float32),
            pltpu.VMEM((NGRP, 8, 8, BPAD), jnp.float32),
            pltpu.VMEM((NGRP, 8, 8, BPAD), jnp.float32),
            pltpu.VMEM((NINSN, D), jnp.float32),
            pltpu.SemaphoreType.DMA,
            pltpu.SemaphoreType.DMA,
            pltpu.SemaphoreType.DMA,
            pltpu.SemaphoreType.DMA,
            pltpu.SemaphoreType.DMA,
            pltpu.SemaphoreType.DMA,
        ],
        compiler_params=pltpu.CompilerParams(
            use_tc_tiling_on_sc=False, needs_layout_passes=False),
    )
    return k(idx_perm, word_table, pos10)


def kernel(word, word_table, pos_table):
    idx_perm = (word.astype(jnp.int32)
                .reshape(NW, NBLK, BBLK, NINSN, INSN)
                .transpose(0, 3, 1, 2, 4)
                .reshape(-1))
    pos10 = lax.slice_in_dim(pos_table, 1, 1 + NINSN, axis=0)
    out5 = _run(idx_perm, word_table, pos10)
    return out5.transpose(2, 4, 0, 1, 3).reshape(B, OUT_PER_ROW, D)


# speedup vs baseline: 1.4857x; 1.0021x over previous
import jax
import jax.numpy as jnp
from jax import lax
from jax.experimental import pallas as pl
from jax.experimental.pallas import tpu as pltpu
from jax.experimental.pallas import tpu_sc as plsc

INSN = 12
NINSN = 10
SEQ = 120
D = 64
NGRP = 3
OUT_PER_ROW = NGRP * NINSN

NC, NS = 2, 16
NW = NC * NS

B = 4096
ROWS_PER_W = B // NW
BBLK = 32
NBLK = ROWS_PER_W // BBLK
GROWS = BBLK * INSN
WIDX = ROWS_PER_W * SEQ
BPAD = 129


def _body(idx_hbm, table_hbm, pos_hbm, out_hbm,
          idx_v0, idx_v1, rows_v0, rows_v1, out_v0, out_v1, pos_v,
          gsem0, gsem1, osem0, osem1, isem0, isem1):
    wid = lax.axis_index("s") * NC + lax.axis_index("c")
    idx_v = (idx_v0, idx_v1)
    rows_v = (rows_v0, rows_v1)
    out_v = (out_v0, out_v1)
    gsem = (gsem0, gsem1)
    osem = (osem0, osem1)
    isem = (isem0, isem1)

    pltpu.sync_copy(pos_hbm, pos_v)
    ibase = wid * WIDX

    def idx_start(p):
        pltpu.async_copy(idx_hbm.at[pl.ds(ibase + p * GROWS, GROWS)],
                         idx_v[p % 2], isem[p % 2])

    def idx_wait(p):
        pltpu.make_async_copy(idx_hbm.at[pl.ds(ibase, GROWS)],
                              idx_v[p % 2], isem[p % 2]).wait()

    i16 = lax.iota(jnp.int32, 16)
    dh_vec = [(q * 16 + i16) >> 3 for q in range(4)]
    dl_vec = [(q * 16 + i16) & 7 for q in range(4)]
    g_vec = [jnp.full((16,), g, jnp.int32) for g in range(NGRP)]

    def gather_start(p):
        pltpu.async_copy(table_hbm.at[idx_v[p % 2]],
                         rows_v[p % 2], gsem[p % 2])

    def gather_wait(p):
        pltpu.make_async_copy(table_hbm.at[idx_v[p % 2]],
                              rows_v[p % 2], gsem[p % 2]).wait()

    def out_dst(j):
        return out_hbm.at[pl.ds(NGRP * j, NGRP), :, wid, :, :]

    pltpu.sync_copy(idx_hbm.at[pl.ds(ibase, GROWS)], idx_v[0])
    idx_start(1)
    gather_start(0)

    for j in range(NINSN):
        oj = j % 2
        ob = out_v[oj]
        if j >= 2:
            pltpu.make_async_copy(ob.at[:, :, :, pl.ds(0, 128)],
                                  out_dst(j), osem[oj]).wait()
        pos_q = [pos_v[j, pl.ds(q * 16, 16)] for q in range(4)]

        for blk in range(NBLK):
            p = j * NBLK + blk
            if p + 1 < NINSN * NBLK:
                idx_wait(p + 1)
                gather_start(p + 1)
            gather_wait(p)
            if p + 2 < NINSN * NBLK:
                idx_start(p + 2)
            rows = rows_v[p % 2]

            @pl.loop(0, BBLK, unroll=2)
            def brow(b32):
                ro = b32 * INSN
                b_bc = jnp.full((16,), blk * BBLK, jnp.int32) + b32
                for q in range(4):
                    sl = pl.ds(q * 16, 16)
                    a1 = rows[ro + 0, sl] + rows[ro + 1, sl]
                    a2 = ((rows[ro + 2, sl] + rows[ro + 3, sl])
                          + (rows[ro + 4, sl] + rows[ro + 5, sl])
                          + rows[ro + 6, sl])
                    a3 = ((rows[ro + 7, sl] + rows[ro + 8, sl])
                          + (rows[ro + 9, sl] + rows[ro + 10, sl])
                          + rows[ro + 11, sl])
                    for g, a in ((0, a1), (1, a2), (2, a3)):
                        plsc.store_scatter(
                            ob, [g_vec[g], dh_vec[q], dl_vec[q], b_bc],
                            a + pos_q[q])

        pltpu.async_copy(ob.at[:, :, :, pl.ds(0, 128)], out_dst(j), osem[oj])

    for oj in range(2):
        pltpu.make_async_copy(out_v[oj].at[:, :, :, pl.ds(0, 128)],
                              out_dst(NINSN - 2 + oj), osem[oj]).wait()


@jax.jit
def _run(idx_perm, word_table, pos10):
    mesh = plsc.VectorSubcoreMesh(
        core_axis_name="c", subcore_axis_name="s", num_cores=NC, num_subcores=NS)
    k = pl.kernel(
        _body,
        out_type=jax.ShapeDtypeStruct((OUT_PER_ROW, 8, NW, 8, 128),
                                      jnp.float32),
        mesh=mesh,
        scratch_types=[
            pltpu.VMEM((GROWS,), jnp.int32),
            pltpu.VMEM((GROWS,), jnp.int32),
            pltpu.VMEM((GROWS, D), jnp.float32),
            pltpu.VMEM((GROWS, D), jnp.float32),
            pltpu.VMEM((NGRP, 8, 8, BPAD), jnp.float32),
            pltpu.VMEM((NGRP, 8, 8, BPAD), jnp.float32),
            pltpu.VMEM((NINSN, D), jnp.float32),
            pltpu.SemaphoreType.DMA,
            pltpu.SemaphoreType.DMA,
            pltpu.SemaphoreType.DMA,
            pltpu.SemaphoreType.DMA,
            pltpu.SemaphoreType.DMA,
            pltpu.SemaphoreType.DMA,
        ],
        compiler_params=pltpu.CompilerParams(
            use_tc_tiling_on_sc=False, needs_layout_passes=False),
    )
    return k(idx_perm, word_table, pos10)


def kernel(word, word_table, pos_table):
    idx_perm = (word.astype(jnp.int32)
                .reshape(NW, NBLK, BBLK, NINSN, INSN)
                .transpose(0, 3, 1, 2, 4)
                .reshape(-1))
    pos10 = lax.slice_in_dim(pos_table, 1, 1 + NINSN, axis=0)
    out5 = _run(idx_perm, word_table, pos10)
    return out5.transpose(2, 4, 0, 1, 3).reshape(B, OUT_PER_ROW, D)
